# baseline (speedup 1.0000x reference)
import functools

import jax
import jax.numpy as jnp
from jax import lax
from jax.experimental import pallas as pl
from jax.experimental.pallas import tpu as pltpu
from jax.experimental.pallas import tpu_sc as plsc

_LANES = 16


@functools.lru_cache(maxsize=None)
def _build_kernels(B, S, D, V):
    info = plsc.get_sparse_core_info()
    NC, NS = info.num_cores, info.num_subcores
    NW = NC * NS
    C = 128
    PK = 128 // D
    VG = V // PK
    NFCH = V // C
    NFULL = NFCH // NW
    NEXTRA = NFCH - NFULL * NW
    NBC = B // C
    BCW = NBC // NW

    mesh = plsc.VectorSubcoreMesh(core_axis_name="c", subcore_axis_name="s")
    cparams = pltpu.CompilerParams(use_tc_tiling_on_sc=True,
                                   needs_layout_passes=False)

    @functools.partial(
        pl.kernel,
        out_type=jax.ShapeDtypeStruct((VG, PK * D), jnp.float32),
        mesh=mesh,
        compiler_params=cparams,
        scratch_types=[
            pltpu.VMEM((2, D, C + 1), jnp.float32),
            pltpu.VMEM((2, C // PK, PK * D), jnp.float32),
            pltpu.SemaphoreType.DMA,
            pltpu.SemaphoreType.DMA,
        ],
    )
    def repack_kernel(tab_t, tail_t, scr, cbuf, sbuf, sem_l, sem_w):
        wid = lax.axis_index("s") * NC + lax.axis_index("c")
        lane = lax.iota(jnp.int32, _LANES)

        def load(c, slot):
            pltpu.async_copy(tab_t.at[:, pl.ds(c * C, C)],
                             cbuf.at[slot, :, pl.ds(0, C)], sem_l)

        def wait_load(slot):
            pltpu.make_async_copy(tab_t.at[:, pl.ds(0, C)],
                                  cbuf.at[slot, :, pl.ds(0, C)],
                                  sem_l).wait()

        def convert(slot, nrl):
            for rl in range(nrl):
                for l0 in range(0, PK * D, _LANES):
                    d_idx = lane + (l0 % D)
                    c_idx = jnp.full((_LANES,), PK * rl + l0 // D, jnp.int32)
                    vec = plsc.load_gather(cbuf.at[slot], [d_idx, c_idx])
                    sbuf[slot, rl, pl.ds(l0, _LANES)] = vec

        def store(c, slot, nrl):
            pltpu.async_copy(sbuf.at[slot, pl.ds(0, nrl)],
                             scr.at[pl.ds(c * (C // PK), nrl)], sem_w)

        def wait_store(nrl):
            pltpu.make_async_copy(sbuf.at[0, pl.ds(0, nrl)],
                                  scr.at[pl.ds(0, nrl)], sem_w).wait()

        load(wid, 0)

        def body(t, carry):
            c = wid + NW * t

            def load_next():
                load(c + NW, (t + 1) % 2)

            pl.when(t + 1 < NFULL)(load_next)
            wait_load(t % 2)
            convert(t % 2, C // PK)

            def drain():
                wait_store(C // PK)

            pl.when(t >= 2)(drain)
            store(c, t % 2, C // PK)
            return carry

        lax.fori_loop(0, NFULL, body, 0)
        wait_store(C // PK)
        wait_store(C // PK)

        ctail = NW * NFULL + wid

        def tail_full():
            load(ctail, 0)
            wait_load(0)
            convert(0, C // PK)
            store(ctail, 0, C // PK)
            wait_store(C // PK)

        def tail_last():
            pltpu.sync_copy(tail_t, cbuf.at[0, :, pl.ds(0, C)])
            convert(0, C // PK)
            pltpu.sync_copy(sbuf.at[0],
                            scr.at[pl.ds(VG - C // PK, C // PK)])

        pl.when(wid < NEXTRA)(tail_full)
        pl.when(wid == NEXTRA)(tail_last)

    NSLOT = 2 + S

    @functools.partial(
        pl.kernel,
        out_type=jax.ShapeDtypeStruct((NSLOT, D, B), jnp.float32),
        mesh=mesh,
        compiler_params=cparams,
        scratch_types=[
            pltpu.VMEM((C,), jnp.int32),
            pltpu.VMEM((2, C), jnp.int32),
            pltpu.VMEM((2, C), jnp.int32),
            pltpu.VMEM((2, C, PK * D + 1), jnp.float32),
            pltpu.VMEM((2, D, C), jnp.float32),
            pltpu.SemaphoreType.DMA,
            pltpu.SemaphoreType.DMA,
        ],
    )
    def gather_kernel(iw, ow, nwf, scr_in, scr_out, out,
                      raw_v, row_v, sub_v, gbuf, obuf, sem_g, sem_w):
        wid = lax.axis_index("s") * NC + lax.axis_index("c")
        bc0 = wid * BCW
        lane = lax.iota(jnp.int32, _LANES)

        def stage_idx(src, off, slot):
            pltpu.sync_copy(src.at[pl.ds(off, C)], raw_v)
            for i in range(C // _LANES):
                raw = raw_v[pl.ds(i * _LANES, _LANES)]
                row_v[slot, pl.ds(i * _LANES, _LANES)] = raw >> (
                    PK.bit_length() - 1)
                sub_v[slot, pl.ds(i * _LANES, _LANES)] = raw & (PK - 1)

        def issue_gather(scr, slot):
            pltpu.async_copy(scr.at[row_v.at[slot]],
                             gbuf.at[slot, :, pl.ds(0, PK * D)], sem_g)

        def wait_gather(scr, slot):
            pltpu.make_async_copy(scr.at[row_v.at[slot]],
                                  gbuf.at[slot, :, pl.ds(0, PK * D)],
                                  sem_g).wait()

        def extract(slot):
            def ex_body(k16, carry):
                k0 = k16 * _LANES
                sub = sub_v[slot, pl.ds(k0, _LANES)]
                k_idx = lane + k0
                col0 = sub * D
                for d in range(D):
                    vec = plsc.load_gather(gbuf.at[slot], [k_idx, col0 + d])
                    obuf[slot, d, pl.ds(k0, _LANES)] = vec
                return carry

            lax.fori_loop(0, C // _LANES, ex_body, 0)

        def write_out(j, bc, slot):
            pltpu.async_copy(obuf.at[slot],
                             out.at[j, :, pl.ds(bc * C, C)], sem_w)

        def wait_write():
            pltpu.make_async_copy(obuf.at[0],
                                  out.at[0, :, pl.ds(0, C)], sem_w).wait()

        def in_unit(t):
            bc = bc0 + t
            return (iw, bc * C, scr_in, 0, bc)

        def out_unit(t):
            bc = bc0 + t
            return (ow, bc * C, scr_out, 1, bc)

        def noise_unit(sidx, t):
            bc = bc0 + t
            return (nwf, sidx * B + bc * C, scr_out, 2 + sidx, bc)

        def run_units(loop_len, unit_of_t):
            src, off, scr, _, _ = unit_of_t(0)
            stage_idx(src, off, 0)
            issue_gather(scr, 0)

            def body(t2, carry):
                for half in range(2):
                    t = t2 * 2 + half
                    slot, nslot = half, 1 - half

                    def prefetch(t=t, nslot=nslot):
                        src2, off2, scr2, _, _ = unit_of_t(t + 1)
                        stage_idx(src2, off2, nslot)
                        issue_gather(scr2, nslot)

                    pl.when(t + 1 < loop_len)(prefetch)
                    _, _, scrl, jl, bcl = unit_of_t(t)
                    wait_gather(scrl, slot)
                    extract(slot)
                    pl.when(t >= 2)(wait_write)
                    write_out(jl, bcl, slot)
                return carry

            lax.fori_loop(0, loop_len // 2, body, 0)
            wait_write()
            wait_write()

        run_units(BCW, in_unit)
        run_units(BCW, out_unit)
        run_units(S * BCW,
                  lambda t: noise_unit(lax.rem(t, S), lax.div(t, S)))

    return repack_kernel, gather_kernel


def kernel(input_words, output_words, noise_words, in_embed, out_embed):
    B, S = noise_words.shape
    V, D = in_embed.shape
    repack, gather = _build_kernels(B, S, D, V)
    scr_in = repack(in_embed.T, in_embed[V - 128:].T)
    scr_out = repack(out_embed.T, out_embed[V - 128:].T)
    nwf = noise_words.T.reshape(S * B)
    out_t = gather(input_words, output_words, nwf, scr_in, scr_out)
    return out_t.transpose(2, 0, 1)


# speedup vs baseline: 1.8669x; 1.8669x over previous
import functools

import jax
import jax.numpy as jnp
from jax import lax
from jax.experimental import pallas as pl
from jax.experimental.pallas import tpu as pltpu
from jax.experimental.pallas import tpu_sc as plsc

_LANES = 16


@functools.lru_cache(maxsize=None)
def _build_kernels(B, S, D, V):
    info = plsc.get_sparse_core_info()
    NC, NS = info.num_cores, info.num_subcores
    NW = NC * NS
    C = 128
    PK = 128 // D
    VG = V // PK
    NFCH = V // C
    NFULL = NFCH // NW
    NEXTRA = NFCH - NFULL * NW
    NBC = B // C
    BCW = NBC // NW

    mesh = plsc.VectorSubcoreMesh(core_axis_name="c", subcore_axis_name="s")
    cparams = pltpu.CompilerParams(use_tc_tiling_on_sc=True,
                                   needs_layout_passes=False)

    @functools.partial(
        pl.kernel,
        out_type=jax.ShapeDtypeStruct((VG, PK * D), jnp.float32),
        mesh=mesh,
        compiler_params=cparams,
        scratch_types=[
            pltpu.VMEM((2, D, C + 1), jnp.float32),
            pltpu.VMEM((2, C // PK, PK * D), jnp.float32),
            pltpu.SemaphoreType.DMA,
            pltpu.SemaphoreType.DMA,
        ],
    )
    def repack_kernel(tab_t, tail_t, scr, cbuf, sbuf, sem_l, sem_w):
        wid = lax.axis_index("s") * NC + lax.axis_index("c")
        lane = lax.iota(jnp.int32, _LANES)

        def load(c, slot):
            pltpu.async_copy(tab_t.at[:, pl.ds(c * C, C)],
                             cbuf.at[slot, :, pl.ds(0, C)], sem_l)

        def wait_load(slot):
            pltpu.make_async_copy(tab_t.at[:, pl.ds(0, C)],
                                  cbuf.at[slot, :, pl.ds(0, C)],
                                  sem_l).wait()

        def convert(slot, nrl):
            for rl in range(nrl):
                for l0 in range(0, PK * D, _LANES):
                    d_idx = lane + (l0 % D)
                    c_idx = jnp.full((_LANES,), PK * rl + l0 // D, jnp.int32)
                    vec = plsc.load_gather(cbuf.at[slot], [d_idx, c_idx])
                    sbuf[slot, rl, pl.ds(l0, _LANES)] = vec

        def store(c, slot, nrl):
            pltpu.async_copy(sbuf.at[slot, pl.ds(0, nrl)],
                             scr.at[pl.ds(c * (C // PK), nrl)], sem_w)

        def wait_store(nrl):
            pltpu.make_async_copy(sbuf.at[0, pl.ds(0, nrl)],
                                  scr.at[pl.ds(0, nrl)], sem_w).wait()

        load(wid, 0)

        def body(t, carry):
            c = wid + NW * t

            def load_next():
                load(c + NW, (t + 1) % 2)

            pl.when(t + 1 < NFULL)(load_next)
            wait_load(t % 2)
            convert(t % 2, C // PK)

            def drain():
                wait_store(C // PK)

            pl.when(t >= 2)(drain)
            store(c, t % 2, C // PK)
            return carry

        lax.fori_loop(0, NFULL, body, 0)
        wait_store(C // PK)
        wait_store(C // PK)

        ctail = NW * NFULL + wid

        def tail_full():
            load(ctail, 0)
            wait_load(0)
            convert(0, C // PK)
            store(ctail, 0, C // PK)
            wait_store(C // PK)

        def tail_last():
            pltpu.sync_copy(tail_t, cbuf.at[0, :, pl.ds(0, C)])
            convert(0, C // PK)
            pltpu.sync_copy(sbuf.at[0],
                            scr.at[pl.ds(VG - C // PK, C // PK)])

        pl.when(wid < NEXTRA)(tail_full)
        pl.when(wid == NEXTRA)(tail_last)

    NSLOT = 2 + S

    @functools.partial(
        pl.kernel,
        out_type=jax.ShapeDtypeStruct((NSLOT, D, B), jnp.float32),
        mesh=mesh,
        compiler_params=cparams,
        scratch_types=[
            pltpu.VMEM((C,), jnp.int32),
            pltpu.VMEM((2, C), jnp.int32),
            pltpu.VMEM((2, C), jnp.int32),
            pltpu.VMEM((2, C, PK * D + 1), jnp.float32),
            pltpu.VMEM((2, D, C), jnp.float32),
            pltpu.SemaphoreType.DMA,
            pltpu.SemaphoreType.DMA,
        ],
    )
    def gather_kernel(iw, ow, nwf, scr_in, scr_out, out,
                      raw_v, row_v, sub_v, gbuf, obuf, sem_g, sem_w):
        wid = lax.axis_index("s") * NC + lax.axis_index("c")
        bc0 = wid * BCW
        lane = lax.iota(jnp.int32, _LANES)

        def stage_idx(src, off, slot):
            pltpu.sync_copy(src.at[pl.ds(off, C)], raw_v)
            for i in range(C // _LANES):
                raw = raw_v[pl.ds(i * _LANES, _LANES)]
                row_v[slot, pl.ds(i * _LANES, _LANES)] = raw >> (
                    PK.bit_length() - 1)
                sub_v[slot, pl.ds(i * _LANES, _LANES)] = raw & (PK - 1)

        def issue_gather(scr, slot):
            pltpu.async_copy(scr.at[row_v.at[slot]],
                             gbuf.at[slot, :, pl.ds(0, PK * D)], sem_g)

        def wait_gather(scr, slot):
            pltpu.make_async_copy(scr.at[row_v.at[slot]],
                                  gbuf.at[slot, :, pl.ds(0, PK * D)],
                                  sem_g).wait()

        def extract(slot):
            def ex_body(k16, carry):
                k0 = k16 * _LANES
                sub = sub_v[slot, pl.ds(k0, _LANES)]
                k_idx = lane + k0
                col0 = sub * D
                for d in range(D):
                    vec = plsc.load_gather(gbuf.at[slot], [k_idx, col0 + d])
                    obuf[slot, d, pl.ds(k0, _LANES)] = vec
                return carry

            lax.fori_loop(0, C // _LANES, ex_body, 0)

        def write_out(j, bc, slot):
            pltpu.async_copy(obuf.at[slot],
                             out.at[j, :, pl.ds(bc * C, C)], sem_w)

        def wait_write():
            pltpu.make_async_copy(obuf.at[0],
                                  out.at[0, :, pl.ds(0, C)], sem_w).wait()

        def in_unit(t):
            bc = bc0 + t
            return (iw, bc * C, scr_in, 0, bc)

        def out_unit(t):
            bc = bc0 + t
            return (ow, bc * C, scr_out, 1, bc)

        def noise_unit(sidx, t):
            bc = bc0 + t
            return (nwf, sidx * B + bc * C, scr_out, 2 + sidx, bc)

        def run_units(loop_len, unit_of_t):
            src, off, scr, _, _ = unit_of_t(0)
            stage_idx(src, off, 0)
            issue_gather(scr, 0)

            def body(t2, carry):
                for half in range(2):
                    t = t2 * 2 + half
                    slot, nslot = half, 1 - half

                    def prefetch(t=t, nslot=nslot):
                        src2, off2, scr2, _, _ = unit_of_t(t + 1)
                        stage_idx(src2, off2, nslot)
                        issue_gather(scr2, nslot)

                    pl.when(t + 1 < loop_len)(prefetch)
                    _, _, scrl, jl, bcl = unit_of_t(t)
                    wait_gather(scrl, slot)
                    extract(slot)
                    pl.when(t >= 2)(wait_write)
                    write_out(jl, bcl, slot)
                return carry

            lax.fori_loop(0, loop_len // 2, body, 0)
            wait_write()
            wait_write()

        run_units(BCW, in_unit)
        run_units(BCW, out_unit)
        run_units(S * BCW,
                  lambda t: noise_unit(lax.rem(t, S), lax.div(t, S)))

    return repack_kernel, gather_kernel


def kernel(input_words, output_words, noise_words, in_embed, out_embed):
    B, S = noise_words.shape
    V, D = in_embed.shape
    repack, gather = _build_kernels(B, S, D, V)
    scr_in = in_embed.reshape(V // (128 // D), 128)
    scr_out = out_embed.reshape(V // (128 // D), 128)
    nwf = noise_words.T.reshape(S * B)
    out_t = gather(input_words, output_words, nwf, scr_in, scr_out)
    return out_t.transpose(2, 0, 1)


# speedup vs baseline: 2.0165x; 1.0801x over previous
import functools

import jax
import jax.numpy as jnp
from jax import lax
from jax.experimental import pallas as pl
from jax.experimental.pallas import tpu as pltpu
from jax.experimental.pallas import tpu_sc as plsc

_LANES = 16


@functools.lru_cache(maxsize=None)
def _build_sc_gather(B, S, D):
    info = plsc.get_sparse_core_info()
    NC, NS = info.num_cores, info.num_subcores
    NW = NC * NS
    NB = B // NW
    C = 128
    K = 4
    NSLOT = 2 + S
    T_IO = NB // C
    T_NZ = (NB * S) // C
    NV = C // _LANES

    mesh = plsc.VectorSubcoreMesh(core_axis_name="c", subcore_axis_name="s")

    @functools.partial(
        pl.kernel,
        out_type=jax.ShapeDtypeStruct((B * NSLOT, D), jnp.float32),
        mesh=mesh,
        compiler_params=pltpu.CompilerParams(use_tc_tiling_on_sc=False),
        scratch_types=[
            pltpu.VMEM((NB,), jnp.int32),
            pltpu.VMEM((NB,), jnp.int32),
            pltpu.VMEM((NB * S,), jnp.int32),
            pltpu.VMEM((K, C), jnp.int32),
            pltpu.VMEM((K, C, D), jnp.float32),
            pltpu.SemaphoreType.DMA,
            pltpu.SemaphoreType.DMA,
        ],
    )
    def sc_kernel(iw_hbm, ow_hbm, nwf_hbm, in_emb, out_emb, out_hbm,
                  idx_in, idx_out, idx_nz, dest_v, rows_v, sem_g, sem_s):
        wid = lax.axis_index("s") * NC + lax.axis_index("c")
        lane = lax.iota(jnp.int32, _LANES)

        pltpu.sync_copy(iw_hbm.at[pl.ds(wid * NB, NB)], idx_in)
        pltpu.sync_copy(ow_hbm.at[pl.ds(wid * NB, NB)], idx_out)
        pltpu.sync_copy(nwf_hbm.at[pl.ds(wid * NB * S, NB * S)], idx_nz)

        def run_phase(T, idx_vmem, table, dest_of):
            def issue(t, slot):
                pltpu.async_copy(table.at[idx_vmem.at[pl.ds(t * C, C)]],
                                 rows_v.at[slot], sem_g)
                for v in range(NV):
                    dest_v[slot, pl.ds(v * _LANES, _LANES)] = \
                        dest_of(t, lane + v * _LANES)

            def wait_gather(slot):
                pltpu.make_async_copy(table.at[idx_vmem.at[pl.ds(0, C)]],
                                      rows_v.at[slot], sem_g).wait()

            def wait_scatter():
                pltpu.make_async_copy(rows_v.at[0],
                                      out_hbm.at[dest_v.at[0]], sem_s).wait()

            for k in range(K - 1):
                issue(k, k)

            def outer(to, carry):
                for k in range(K):
                    t = to * K + k
                    wait_gather(k)
                    if k == 0:
                        pl.when(to > 0)(wait_scatter)
                    else:
                        wait_scatter()
                    pltpu.async_copy(rows_v.at[k],
                                     out_hbm.at[dest_v.at[k]], sem_s)
                    nxt = t + K - 1

                    def issue_nxt(nxt=nxt, slot=(k - 1) % K):
                        issue(nxt, slot)

                    pl.when(nxt < T)(issue_nxt)
                return carry

            lax.fori_loop(0, T // K, outer, 0)
            wait_scatter()

        gb0 = wid * NB

        run_phase(T_IO, idx_in, in_emb,
                  lambda t, i: (gb0 + t * C + i) * NSLOT)
        run_phase(T_IO, idx_out, out_emb,
                  lambda t, i: (gb0 + t * C + i) * NSLOT + 1)

        def dest_noise(t, i):
            n = gb0 * S + t * C + i
            q = (n.astype(jnp.float32) * (1.0 / S)).astype(jnp.int32)
            r = n - q * S
            q = q + jnp.where(r >= S, 1, 0) - jnp.where(r < 0, 1, 0)
            return n + 2 + 2 * q

        run_phase(T_NZ, idx_nz, out_emb, dest_noise)

    return sc_kernel


def kernel(input_words, output_words, noise_words, in_embed, out_embed):
    B, S = noise_words.shape
    D = in_embed.shape[1]
    sc = _build_sc_gather(B, S, D)
    out_flat = sc(input_words, output_words, noise_words.reshape(B * S),
                  in_embed, out_embed)
    return out_flat.reshape(B, 2 + S, D)
